# baseline (speedup 1.0000x reference)
import functools

import jax
import jax.numpy as jnp
from jax.experimental import pallas as pl


def _agg_kernel(len_ref, x_ref, wq_ref, out_ref, *, b_block: int):
    t = x_ref.shape[1]
    d = x_ref.shape[2]
    t_idx = jax.lax.broadcasted_iota(jnp.int32, (1, t), 1)
    mask = (t_idx < len_ref[...]).astype(jnp.float32)
    wq_row = wq_ref[...].reshape(1, d)
    for bb in range(b_block):
        xb = x_ref[bb]
        w = jnp.sum(xb * wq_row, axis=1)
        a = mask * w.reshape(1, t)
        out_ref[bb] = jax.lax.dot_general(
            a, xb, (((1,), (0,)), ((), ())),
            preferred_element_type=jnp.float32)


def kernel(lengths, nodes_output, Wq):
    B, T, D = nodes_output.shape
    n_len = lengths.shape[0]
    b_block = 2
    lens = jnp.asarray(lengths, dtype=jnp.int32).reshape(n_len, 1)

    grid = (B // b_block,)
    out = pl.pallas_call(
        functools.partial(_agg_kernel, b_block=b_block),
        grid=grid,
        in_specs=[
            pl.BlockSpec((n_len, 1), lambda g: (0, 0)),
            pl.BlockSpec((b_block, T, D), lambda g: (g, 0, 0)),
            pl.BlockSpec((D, 1), lambda g: (0, 0)),
        ],
        out_specs=pl.BlockSpec((b_block, n_len, D), lambda g: (g, 0, 0)),
        out_shape=jax.ShapeDtypeStruct((B, n_len, D), jnp.float32),
    )(lens, nodes_output, Wq)
    return out.transpose(1, 0, 2).reshape(n_len * B, D)


# speedup vs baseline: 1.0517x; 1.0517x over previous
import functools

import jax
import jax.numpy as jnp
from jax.experimental import pallas as pl


def _agg_kernel(len_ref, x_ref, wq_ref, out_ref, *, b_block: int):
    t = x_ref.shape[1]
    d = x_ref.shape[2]
    t_idx = jax.lax.broadcasted_iota(jnp.int32, (1, t), 1)
    mask = (t_idx < len_ref[...]).astype(jnp.float32)
    wq_row = wq_ref[...].reshape(1, d)
    for bb in range(b_block):
        xb = x_ref[bb]
        w = jnp.sum(xb * wq_row, axis=1)
        a = mask * w.reshape(1, t)
        out_ref[bb] = jax.lax.dot_general(
            a, xb, (((1,), (0,)), ((), ())),
            preferred_element_type=jnp.float32)


def kernel(lengths, nodes_output, Wq):
    B, T, D = nodes_output.shape
    n_len = lengths.shape[0]
    b_block = 1
    lens = jnp.asarray(lengths, dtype=jnp.int32).reshape(n_len, 1)

    grid = (B // b_block,)
    out = pl.pallas_call(
        functools.partial(_agg_kernel, b_block=b_block),
        grid=grid,
        in_specs=[
            pl.BlockSpec((n_len, 1), lambda g: (0, 0)),
            pl.BlockSpec((b_block, T, D), lambda g: (g, 0, 0)),
            pl.BlockSpec((D, 1), lambda g: (0, 0)),
        ],
        out_specs=pl.BlockSpec((b_block, n_len, D), lambda g: (g, 0, 0)),
        out_shape=jax.ShapeDtypeStruct((B, n_len, D), jnp.float32),
    )(lens, nodes_output, Wq)
    return out.transpose(1, 0, 2).reshape(n_len * B, D)
